# Initial kernel scaffold; baseline (speedup 1.0000x reference)
import jax
import jax.numpy as jnp
from jax.experimental import pallas as pl


def kernel(x):
    raise NotImplementedError("write your pallas kernel here")


# speedup vs baseline: 1.2025x; 1.2025x over previous
import functools

import jax
import jax.numpy as jnp
from jax import lax
from jax.experimental import pallas as pl
from jax.experimental.pallas import tpu as pltpu
from jax.experimental.pallas import tpu_sc as plsc

B = 16
N = 2048
C = 512
NWIN = 16
W = 16
STRIDE = 128
L = 16
NC = 2
NS = 16
NW = NC * NS
NPAIR = B * NWIN
PPW = NPAIR // NW


def _body(x_hbm, out_hbm, rows_v, acc_v, sem0, sem1):
    wid = lax.axis_index("s") * NC + lax.axis_index("c")
    base = wid * PPW
    sems = (sem0, sem1)

    def row_start(k):
        p = base + k
        b = p // NWIN
        i = p - b * NWIN
        return b * N + i * STRIDE

    def start(k):
        return pltpu.async_copy(
            x_hbm.at[pl.ds(row_start(k), W)],
            rows_v.at[k % 2],
            sems[k % 2],
        )

    def compute(k):
        buf = k % 2

        def chunk(c, carry):
            off = c * L
            s = rows_v[buf, 0, pl.ds(off, L)]
            for r in range(1, W):
                s = s + rows_v[buf, r, pl.ds(off, L)]
            acc_v[k, pl.ds(off, L)] = s * (1.0 / W)
            return carry

        lax.fori_loop(0, C // L, chunk, 0)

    pending = start(0)
    for k in range(PPW):
        nxt = start(k + 1) if k + 1 < PPW else None
        pending.wait()
        compute(k)
        pending = nxt

    pltpu.sync_copy(acc_v, out_hbm.at[pl.ds(base, PPW)])


def kernel(x):
    x_flat = x.reshape(B * N, C)
    mesh = plsc.VectorSubcoreMesh(core_axis_name="c", subcore_axis_name="s")
    run = functools.partial(
        pl.kernel,
        mesh=mesh,
        out_type=jax.ShapeDtypeStruct((NPAIR, C), jnp.float32),
        scratch_types=[
            pltpu.VMEM((2, W, C), jnp.float32),
            pltpu.VMEM((PPW, C), jnp.float32),
            pltpu.SemaphoreType.DMA,
            pltpu.SemaphoreType.DMA,
        ],
    )(_body)
    out = run(x_flat)
    return out.reshape(B, NWIN, C)


# speedup vs baseline: 1.2318x; 1.0243x over previous
import functools

import jax
import jax.numpy as jnp
from jax import lax
from jax.experimental import pallas as pl
from jax.experimental.pallas import tpu as pltpu
from jax.experimental.pallas import tpu_sc as plsc

B = 16
N = 2048
C = 512
NWIN = 16
W = 16
STRIDE = 128
L = 16
NC = 2
NS = 16
NW = NC * NS
NPAIR = B * NWIN
PPW = NPAIR // NW


def _body(x_hbm, out_hbm, rows_v, acc_v, sem):
    wid = lax.axis_index("s") * NC + lax.axis_index("c")
    base = wid * PPW

    def row_start(k):
        p = base + k
        b = p // NWIN
        i = p - b * NWIN
        return b * N + i * STRIDE

    handles = [
        pltpu.async_copy(
            x_hbm.at[pl.ds(row_start(k), W)],
            rows_v.at[k],
            sem,
        )
        for k in range(PPW)
    ]

    def compute(k):
        def chunk(c, carry):
            off = c * L
            vals = [rows_v[k, r, pl.ds(off, L)] for r in range(W)]
            while len(vals) > 1:
                nxt = [vals[i] + vals[i + 1] for i in range(0, len(vals) - 1, 2)]
                if len(vals) % 2:
                    nxt.append(vals[-1])
                vals = nxt
            acc_v[k, pl.ds(off, L)] = vals[0] * (1.0 / W)
            return carry

        lax.fori_loop(0, C // L, chunk, 0, unroll=2)

    for k in range(PPW):
        handles[k].wait()
        compute(k)

    pltpu.sync_copy(acc_v, out_hbm.at[pl.ds(base, PPW)])


def kernel(x):
    x_flat = x.reshape(B * N, C)
    mesh = plsc.VectorSubcoreMesh(core_axis_name="c", subcore_axis_name="s")
    run = functools.partial(
        pl.kernel,
        mesh=mesh,
        out_type=jax.ShapeDtypeStruct((NPAIR, C), jnp.float32),
        scratch_types=[
            pltpu.VMEM((PPW, W, C), jnp.float32),
            pltpu.VMEM((PPW, C), jnp.float32),
            pltpu.SemaphoreType.DMA,
        ],
    )(_body)
    out = run(x_flat)
    return out.reshape(B, NWIN, C)


# speedup vs baseline: 1.2699x; 1.0310x over previous
import functools

import jax
import jax.numpy as jnp
from jax import lax
from jax.experimental import pallas as pl
from jax.experimental.pallas import tpu as pltpu
from jax.experimental.pallas import tpu_sc as plsc

B = 16
N = 2048
C = 512
NWIN = 16
W = 16
STRIDE = 128
L = 16
NC = 2
NS = 16
NW = NC * NS
NPAIR = B * NWIN
PPW = NPAIR // NW


def _body(x_hbm, out_hbm, rows_v, acc_v, sem):
    wid = lax.axis_index("s") * NC + lax.axis_index("c")
    base = wid * PPW

    def row_start(k):
        p = base + k
        b = p // NWIN
        i = p - b * NWIN
        return b * N + i * STRIDE

    def fire(k):
        pltpu.async_copy(
            x_hbm.at[pl.ds(row_start(k), W)],
            rows_v.at[lax.rem(k, 2)],
            sem,
        )

    fire(0)

    def pair_body(k, carry):
        @pl.when(k + 1 < PPW)
        def _():
            fire(k + 1)

        buf = lax.rem(k, 2)
        pltpu.make_async_copy(
            x_hbm.at[pl.ds(0, W)], rows_v.at[buf], sem
        ).wait()

        def chunk(c, cr):
            off = c * L
            vals = [rows_v[buf, r, pl.ds(off, L)] for r in range(W)]
            while len(vals) > 1:
                nxt = [vals[i] + vals[i + 1] for i in range(0, len(vals) - 1, 2)]
                if len(vals) % 2:
                    nxt.append(vals[-1])
                vals = nxt
            acc_v[k, pl.ds(off, L)] = vals[0] * (1.0 / W)
            return cr

        lax.fori_loop(0, C // L, chunk, 0)
        return carry

    lax.fori_loop(0, PPW, pair_body, 0)
    pltpu.sync_copy(acc_v, out_hbm.at[pl.ds(base, PPW)])


def kernel(x):
    x_flat = x.reshape(B * N, C)
    mesh = plsc.VectorSubcoreMesh(core_axis_name="c", subcore_axis_name="s")
    run = functools.partial(
        pl.kernel,
        mesh=mesh,
        out_type=jax.ShapeDtypeStruct((NPAIR, C), jnp.float32),
        scratch_types=[
            pltpu.VMEM((2, W, C), jnp.float32),
            pltpu.VMEM((PPW, C), jnp.float32),
            pltpu.SemaphoreType.DMA,
        ],
    )(_body)
    out = run(x_flat)
    return out.reshape(B, NWIN, C)
